# Initial kernel scaffold; baseline (speedup 1.0000x reference)
import functools

import jax
import jax.numpy as jnp
from jax import lax
from jax.experimental import pallas as pl
from jax.experimental.pallas import tpu as pltpu
from jax.experimental.pallas import tpu_sc as plsc

B = 16384
F = 26
H = 16
OUT = F + H
NC, NS, L = 2, 16, 16
NW = NC * NS
BPW = B // NW
C = 128
NCHUNK = BPW // C

_mesh = plsc.VectorSubcoreMesh(core_axis_name="c", subcore_axis_name="s")


@functools.partial(
    pl.kernel,
    out_type=jax.ShapeDtypeStruct((B * OUT,), jnp.float32),
    mesh=_mesh,
    scratch_types=[
        pltpu.VMEM((C * F,), jnp.int32),
        pltpu.VMEM((C * F,), jnp.float32),
        pltpu.VMEM((C * F, H), jnp.float32),
        pltpu.VMEM((C * F,), jnp.float32),
        pltpu.VMEM((C * OUT,), jnp.float32),
        pltpu.SemaphoreType.DMA,
        pltpu.SemaphoreType.DMA,
    ],
)
def _fm_sc(val_hbm, idx_hbm, tab_hbm, wtab_hbm, out_hbm,
           idx_v, val_v, rows_v, w_v, out_v, sem_e, sem_w):
    wid = lax.axis_index("s") * NC + lax.axis_index("c")
    iota = lax.iota(jnp.int32, L)
    for chunk in range(NCHUNK):
        pair0 = (wid * BPW + chunk * C) * F
        pltpu.sync_copy(idx_hbm.at[pl.ds(pair0, C * F)], idx_v)
        pltpu.sync_copy(val_hbm.at[pl.ds(pair0, C * F)], val_v)
        cp_e = pltpu.async_copy(tab_hbm.at[idx_v], rows_v, sem_e)
        cp_w = pltpu.async_copy(wtab_hbm.at[idx_v], w_v, sem_w)
        cp_e.wait()
        cp_w.wait()

        def fm1_body(i, _):
            p0 = i * L
            w16 = w_v[pl.ds(p0, L)]
            v16 = val_v[pl.ds(p0, L)]
            pidx = p0 + iota
            oidx = (pidx // F) * OUT + (pidx % F)
            plsc.store_scatter(out_v, [oidx], w16 * v16)
            return 0

        lax.fori_loop(0, (C * F) // L, fm1_body, 0)

        def fm2_body(b, _):
            acc = jnp.zeros((L,), jnp.float32)
            acc2 = jnp.zeros((L,), jnp.float32)
            for f in range(F):
                p = b * F + f
                ep = rows_v[p, :] * val_v[p]
                acc = acc + ep
                acc2 = acc2 + ep * ep
            out_v[pl.ds(b * OUT + F, H)] = 0.5 * (acc * acc - acc2)
            return 0

        lax.fori_loop(0, C, fm2_body, 0)
        pltpu.sync_copy(out_v,
                        out_hbm.at[pl.ds((wid * BPW + chunk * C) * OUT, C * OUT)])


def kernel(feat_value, feat_index, embed_table, fm_1_weight_table):
    idx = feat_index.astype(jnp.int32).reshape(-1)
    val = feat_value.reshape(-1)
    out = _fm_sc(val, idx, embed_table, fm_1_weight_table)
    return out.reshape(B, OUT)


# speedup vs baseline: 1.1748x; 1.1748x over previous
import functools

import jax
import jax.numpy as jnp
from jax import lax
from jax.experimental import pallas as pl
from jax.experimental.pallas import tpu as pltpu
from jax.experimental.pallas import tpu_sc as plsc

B = 16384
F = 26
H = 16
OUT = F + H
NC, NS, L = 2, 16, 16
NW = NC * NS
BPW = B // NW
C = 128
NCHUNK = BPW // C

_mesh = plsc.VectorSubcoreMesh(core_axis_name="c", subcore_axis_name="s")


@functools.partial(
    pl.kernel,
    out_type=jax.ShapeDtypeStruct((B * OUT,), jnp.float32),
    mesh=_mesh,
    compiler_params=pltpu.CompilerParams(needs_layout_passes=False,
                                         use_tc_tiling_on_sc=False),
    scratch_types=[
        pltpu.VMEM((C * F,), jnp.int32),
        pltpu.VMEM((C * F,), jnp.float32),
        pltpu.VMEM((C * F, H), jnp.float32),
        pltpu.VMEM((C * F,), jnp.float32),
        pltpu.VMEM((C * OUT,), jnp.float32),
        pltpu.SemaphoreType.DMA,
        pltpu.SemaphoreType.DMA,
    ],
)
def _fm_sc(val_hbm, idx_hbm, tab_hbm, wtab_hbm, out_hbm,
           idx_v, val_v, rows_v, w_v, out_v, sem_e, sem_w):
    wid = lax.axis_index("s") * NC + lax.axis_index("c")
    iota = lax.iota(jnp.int32, L)
    for chunk in range(NCHUNK):
        pair0 = (wid * BPW + chunk * C) * F
        pltpu.sync_copy(idx_hbm.at[pl.ds(pair0, C * F)], idx_v)
        pltpu.sync_copy(val_hbm.at[pl.ds(pair0, C * F)], val_v)
        cp_e = pltpu.async_copy(tab_hbm.at[idx_v], rows_v, sem_e)
        cp_w = pltpu.async_copy(wtab_hbm.at[idx_v], w_v, sem_w)
        cp_e.wait()
        cp_w.wait()

        def fm1_body(i, _):
            p0 = i * L
            w16 = w_v[pl.ds(p0, L)]
            v16 = val_v[pl.ds(p0, L)]
            pidx = p0 + iota
            oidx = (pidx // F) * OUT + (pidx % F)
            plsc.store_scatter(out_v, [oidx], w16 * v16)
            return 0

        lax.fori_loop(0, (C * F) // L, fm1_body, 0)

        def fm2_body(b, _):
            acc = jnp.zeros((L,), jnp.float32)
            acc2 = jnp.zeros((L,), jnp.float32)
            for f in range(F):
                p = b * F + f
                vs = plsc.load_gather(val_v, [jnp.full((L,), p, jnp.int32)])
                ep = rows_v[p, :] * vs
                acc = acc + ep
                acc2 = acc2 + ep * ep
            out_v[pl.ds(b * OUT + F, H)] = 0.5 * (acc * acc - acc2)
            return 0

        lax.fori_loop(0, C, fm2_body, 0)
        pltpu.sync_copy(out_v,
                        out_hbm.at[pl.ds((wid * BPW + chunk * C) * OUT, C * OUT)])


def kernel(feat_value, feat_index, embed_table, fm_1_weight_table):
    idx = feat_index.astype(jnp.int32).reshape(-1)
    val = feat_value.reshape(-1)
    out = _fm_sc(val, idx, embed_table, fm_1_weight_table)
    return out.reshape(B, OUT)
